# Initial kernel scaffold; baseline (speedup 1.0000x reference)
import jax
import jax.numpy as jnp
from jax.experimental import pallas as pl


def kernel(y, Wr0, W1_0, b1_0, W2_0, b2_0, Wr1, W1_1, b1_1, W2_1, b2_1):
    raise NotImplementedError("write your pallas kernel here")


# speedup vs baseline: 1.4399x; 1.4399x over previous
import functools

import jax
import jax.numpy as jnp
from jax import lax
from jax.experimental import pallas as pl
from jax.experimental.pallas import tpu as pltpu
from jax.experimental.pallas import tpu_sc as plsc

T = 4096
D = 768
F = 1024
E = 64
C = 80
NSLOT = E * C
PAD = 8
BIG = 1 << 20

NC = 2
NS = 16
NW = NC * NS
EPW = E // NW
SPW = EPW * C
TPW = T // NW
GCH = 32

_f32 = jnp.float32
_i32 = jnp.int32


def _router_body(scale_norm, x_ref, *rest):
    if scale_norm:
        gkp_ref = rest[0]
        wr_ref = rest[1]
        outs = rest[2:]
    else:
        wr_ref = rest[0]
        outs = rest[1:]
    xpad_ref, slotd_ref, slotc_ref, gk_ref, lb_ref, z_ref = outs

    x = x_ref[...]
    if scale_norm:
        x = x * gkp_ref[...]
        ss = jnp.sum(x * x, axis=1, keepdims=True)
        x = x / jnp.maximum(jnp.sqrt(ss), 1e-12)
    xpad_ref[0:T, :] = x
    xpad_ref[T:T + PAD, :] = jnp.zeros((PAD, D), _f32)

    logits = jnp.dot(x, wr_ref[...], preferred_element_type=_f32)
    m = jnp.max(logits, axis=1, keepdims=True)
    ex = jnp.exp(logits - m)
    se = jnp.sum(ex, axis=1, keepdims=True)
    probs = ex / se
    gate = jnp.max(probs, axis=1, keepdims=True)
    ismax = probs >= gate

    rr = lax.broadcasted_iota(_i32, (E, E), 0)
    cc = lax.broadcasted_iota(_i32, (E, E), 1)
    u_incl = (rr <= cc).astype(_f32)
    ismax_f = ismax.astype(_f32)
    cnt = jnp.dot(ismax_f, u_incl, preferred_element_type=_f32)
    sel = jnp.where(ismax & (cnt == 1.0), 1.0, 0.0)

    iota_e = lax.broadcasted_iota(_i32, (1, E), 1).astype(_f32)
    eidx_f = jnp.sum(sel * iota_e, axis=1, keepdims=True)

    br = lax.broadcasted_iota(_i32, (128, 128), 0)
    bc = lax.broadcasted_iota(_i32, (128, 128), 1)
    l_incl = (br >= bc).astype(_f32)
    carry = jnp.zeros((1, E), _f32)
    for b in range(T // 128):
        sb = sel[b * 128:(b + 1) * 128, :]
        s_in = jnp.dot(l_incl, sb, preferred_element_type=_f32)
        posf = s_in + carry - 1.0
        pos_t = jnp.sum(posf * sb, axis=1, keepdims=True)
        e_t = eidx_f[b * 128:(b + 1) * 128, :]
        g_t = gate[b * 128:(b + 1) * 128, :]
        keep = pos_t < float(C)
        slotf = e_t * float(C) + jnp.minimum(pos_t, float(C - 1))
        slotd_ref[b * 128:(b + 1) * 128, :] = jnp.where(
            keep, slotf, float(BIG)).astype(_i32)
        slotc_ref[b * 128:(b + 1) * 128, :] = jnp.where(
            keep, slotf, 0.0).astype(_i32)
        gk_ref[b * 128:(b + 1) * 128, :] = jnp.where(keep, g_t, 0.0)
        carry = carry + s_in[127:128, :]

    f = jnp.mean(sel, axis=0)
    p_mean = jnp.mean(probs, axis=0)
    lb_ref[...] = jnp.reshape(float(E) * jnp.sum(f * p_mean), (1, 1))
    lse = jnp.log(se) + m
    z_ref[...] = jnp.reshape(jnp.mean(lse * lse), (1, 1))


def _make_router(scale_norm):
    out_shape = (
        jax.ShapeDtypeStruct((T + PAD, D), _f32),
        jax.ShapeDtypeStruct((T, 1), _i32),
        jax.ShapeDtypeStruct((T, 1), _i32),
        jax.ShapeDtypeStruct((T, 1), _f32),
        jax.ShapeDtypeStruct((1, 1), _f32),
        jax.ShapeDtypeStruct((1, 1), _f32),
    )
    return pl.pallas_call(
        functools.partial(_router_body, scale_norm),
        out_shape=out_shape,
    )


_router0 = _make_router(False)
_router1 = _make_router(True)


def _dispatch_body(xpad_hbm, slotd_hbm, buf_hbm, slotd_v, islot_v, rows_v,
                   sem):
    cid = lax.axis_index("c")
    sid = lax.axis_index("s")
    wid = sid * NC + cid
    lo = wid * SPW

    pltpu.sync_copy(slotd_hbm, slotd_v)
    for j in range(SPW // 16):
        islot_v[pl.ds(j * 16, 16)] = jnp.full((16,), T, _i32)

    iota16 = lax.iota(_i32, 16)

    def scan_chunk(i, carry):
        sl = slotd_v[pl.ds(i * 16, 16)]
        rel = sl - lo
        msk = (rel >= 0) & (rel < SPW)
        tok = iota16 + i * 16
        plsc.store_scatter(islot_v, [rel], tok, mask=msk)
        return carry

    lax.fori_loop(0, T // 16, scan_chunk, 0)

    for j in range(SPW // GCH):
        cp = pltpu.async_copy(
            xpad_hbm.at[islot_v.at[pl.ds(j * GCH, GCH)]], rows_v, sem)
        cp.wait()
        pltpu.sync_copy(rows_v, buf_hbm.at[pl.ds(lo + j * GCH, GCH)])


@functools.cache
def _get_dispatch():
    return pl.kernel(
        _dispatch_body,
        out_type=jax.ShapeDtypeStruct((NSLOT, D), _f32),
        mesh=plsc.VectorSubcoreMesh(core_axis_name="c",
                                    subcore_axis_name="s"),
        compiler_params=pltpu.CompilerParams(needs_layout_passes=False),
        scratch_types=[
            pltpu.VMEM((T,), _i32),
            pltpu.VMEM((SPW,), _i32),
            pltpu.VMEM((GCH, D), _f32),
            pltpu.SemaphoreType.DMA,
        ],
    )


def _combine_body(eo_hbm, slotc_hbm, raw_hbm, idx_v, rows_v, sem):
    cid = lax.axis_index("c")
    sid = lax.axis_index("s")
    wid = sid * NC + cid
    base = wid * TPW

    pltpu.sync_copy(slotc_hbm.at[pl.ds(base, TPW)], idx_v)
    for j in range(TPW // GCH):
        cp = pltpu.async_copy(
            eo_hbm.at[idx_v.at[pl.ds(j * GCH, GCH)]], rows_v, sem)
        cp.wait()
        pltpu.sync_copy(rows_v, raw_hbm.at[pl.ds(base + j * GCH, GCH)])


@functools.cache
def _get_combine():
    return pl.kernel(
        _combine_body,
        out_type=jax.ShapeDtypeStruct((T, D), _f32),
        mesh=plsc.VectorSubcoreMesh(core_axis_name="c",
                                    subcore_axis_name="s"),
        compiler_params=pltpu.CompilerParams(needs_layout_passes=False),
        scratch_types=[
            pltpu.VMEM((TPW,), _i32),
            pltpu.VMEM((GCH, D), _f32),
            pltpu.SemaphoreType.DMA,
        ],
    )


def _ffn_body(buf_ref, w1_ref, b1_ref, w2_ref, b2_ref, eo_ref):
    x = buf_ref[0]
    h = jnp.dot(x, w1_ref[0], preferred_element_type=_f32) + b1_ref[0]
    h = jnp.maximum(h, 0.0)
    eo = jnp.dot(h, w2_ref[0], preferred_element_type=_f32) + b2_ref[0]
    eo_ref[0, :, :] = eo


_ffn = pl.pallas_call(
    _ffn_body,
    grid=(E,),
    in_specs=[
        pl.BlockSpec((1, C, D), lambda e: (e, 0, 0)),
        pl.BlockSpec((1, D, F), lambda e: (e, 0, 0)),
        pl.BlockSpec((1, 1, F), lambda e: (e, 0, 0)),
        pl.BlockSpec((1, F, D), lambda e: (e, 0, 0)),
        pl.BlockSpec((1, 1, D), lambda e: (e, 0, 0)),
    ],
    out_specs=pl.BlockSpec((1, C, D), lambda e: (e, 0, 0)),
    out_shape=jax.ShapeDtypeStruct((E, C, D), _f32),
)


def _final_body(raw_ref, gk_ref, out_ref):
    x = raw_ref[...] * gk_ref[...]
    ss = jnp.sum(x * x, axis=1, keepdims=True)
    out_ref[...] = x / jnp.maximum(jnp.sqrt(ss), 1e-12)


_finalize = pl.pallas_call(
    _final_body,
    out_shape=jax.ShapeDtypeStruct((T, D), _f32),
)


def kernel(y, Wr0, W1_0, b1_0, W2_0, b2_0, Wr1, W1_1, b1_1, W2_1, b2_1):
    dispatch, combine = _get_dispatch(), _get_combine()
    xpad0, slotd0, slotc0, gk0, lb0, z0 = _router0(y, Wr0)
    buf0 = dispatch(xpad0, slotd0.reshape(T))
    eo0 = _ffn(buf0.reshape(E, C, D), W1_0, b1_0.reshape(E, 1, F),
               W2_0, b2_0.reshape(E, 1, D))
    h0raw = combine(eo0.reshape(NSLOT, D), slotc0.reshape(T))

    xpad1, slotd1, slotc1, gk1, lb1, z1 = _router1(h0raw, gk0, Wr1)
    buf1 = dispatch(xpad1, slotd1.reshape(T))
    eo1 = _ffn(buf1.reshape(E, C, D), W1_1, b1_1.reshape(E, 1, F),
               W2_1, b2_1.reshape(E, 1, D))
    h1raw = combine(eo1.reshape(NSLOT, D), slotc1.reshape(T))

    out = _finalize(h1raw, gk1)
    return (out, lb0.reshape(()), z0.reshape(()), lb1.reshape(()),
            z1.reshape(()))
